# baseline (speedup 1.0000x reference)
import functools

import jax
import jax.numpy as jnp
import numpy as np
from jax import lax
from jax.experimental import pallas as pl
from jax.experimental.pallas import tpu as pltpu
from jax.experimental.pallas import tpu_sc as plsc

_N = 10000
_E = 320000
_D = 128
_DP = 64
_NPAD = 10112
_EPAD = 331776
_EPW = _EPAD // 32
_CH = 128
_NCHT = _EPW // _CH
_NCHA = 40
_RPT = _NPAD // 16

_NC, _NS = 2, 16

_NDUM = _EPAD - _E - _N
_TAIL_SRC = np.concatenate([
    np.arange(_N, dtype=np.int32),
    (np.arange(_NDUM, dtype=np.int32) * 7) % _N,
])
_TAIL_DST = np.concatenate([
    np.arange(_N, dtype=np.int32),
    _N + np.arange(_NDUM, dtype=np.int32) % (_NPAD - _N),
])
_ZERO1 = np.zeros((_NPAD,), np.float32)
_ZERO2 = np.zeros((_NPAD, _D), np.float32)


def _mesh():
    return plsc.VectorSubcoreMesh(core_axis_name="c", subcore_axis_name="s")


def _deg_body(dst_hbm, zero_hbm, out_hbm, didx, ones_v, stg, dacc, sem):
    c = lax.axis_index("c")
    s = lax.axis_index("s")
    wid = c * _NS + s
    pltpu.sync_copy(zero_hbm.at[pl.ds(s * _RPT, _RPT)], stg)
    pltpu.sync_copy(stg, dacc.at[pl.ds(s * _RPT, _RPT)])
    pltpu.sync_copy(dst_hbm.at[wid], didx)
    for i in range(_CH // 16):
        ones_v[pl.ds(i * 16, 16)] = jnp.ones((16,), jnp.float32)
    plsc.subcore_barrier()

    def body(j, carry):
        pltpu.async_copy(ones_v, dacc.at[didx.at[j]], sem, add=True)
        return carry

    lax.fori_loop(0, _NCHT, body, 0)

    def drain(j, carry):
        pltpu.make_async_copy(ones_v, dacc.at[didx.at[j]], sem).wait()
        return carry

    lax.fori_loop(0, _NCHT, drain, 0)
    plsc.subcore_barrier()
    pltpu.sync_copy(dacc.at[pl.ds(s * _RPT, _RPT)], stg)
    pltpu.sync_copy(stg, out_hbm.at[pl.ds(c * _NPAD + s * _RPT, _RPT)])


@functools.partial(
    pl.kernel,
    mesh=_mesh(),
    out_type=jax.ShapeDtypeStruct((_NC * _NPAD,), jnp.float32),
    scratch_types=[
        pltpu.VMEM((_NCHT, _CH), jnp.int32),
        pltpu.VMEM((_CH,), jnp.float32),
        pltpu.VMEM((_RPT,), jnp.float32),
        pltpu.VMEM_SHARED((_NPAD,), jnp.float32),
        pltpu.SemaphoreType.DMA,
    ],
)
def _deg_sc(dst_hbm, zero_hbm, out_hbm, didx, ones_v, stg, dacc, sem):
    _deg_body(dst_hbm, zero_hbm, out_hbm, didx, ones_v, stg, dacc, sem)


def _agg_body(y_hbm, src_hbm, dst_hbm, zero_hbm, out_hbm,
              sidx, didx, r0, r1, acc,
              sg0, sg1, ss0, ss1):
    c = lax.axis_index("c")
    s = lax.axis_index("s")
    wid = c * _NS + s
    pltpu.sync_copy(zero_hbm.at[pl.ds(s * _RPT, _CH)], r0)
    nz = _RPT // _CH
    rem = _RPT - nz * _CH
    for k in range(nz):
        pltpu.async_copy(r0, acc.at[pl.ds(s * _RPT + k * _CH, _CH)], sg0)
    if rem:
        pltpu.async_copy(r0.at[pl.ds(0, rem)],
                         acc.at[pl.ds(s * _RPT + nz * _CH, rem)], sg0)
    pltpu.async_copy(src_hbm.at[wid, pl.ds(0, _NCHA)],
                     sidx.at[pl.ds(0, _NCHA)], sg1)
    pltpu.async_copy(dst_hbm.at[wid, pl.ds(0, _NCHA)],
                     didx.at[pl.ds(0, _NCHA)], sg1)
    for k in range(nz):
        pltpu.make_async_copy(r0, acc.at[pl.ds(s * _RPT + k * _CH, _CH)],
                              sg0).wait()
    if rem:
        pltpu.make_async_copy(r0.at[pl.ds(0, rem)],
                              acc.at[pl.ds(s * _RPT + nz * _CH, rem)],
                              sg0).wait()
    pltpu.make_async_copy(src_hbm.at[wid, pl.ds(0, _NCHA)],
                          sidx.at[pl.ds(0, _NCHA)], sg1).wait()
    pltpu.make_async_copy(dst_hbm.at[wid, pl.ds(0, _NCHA)],
                          didx.at[pl.ds(0, _NCHA)], sg1).wait()
    plsc.subcore_barrier()

    def run(n):
        def wait_g(buf, sem, r):
            pltpu.make_async_copy(y_hbm.at[sidx.at[r]], buf, sem).wait()

        def wait_s(buf, sem, r):
            pltpu.make_async_copy(buf, acc.at[didx.at[r]], sem).wait()

        pltpu.async_copy(y_hbm.at[sidx.at[0]], r0, sg0)

        def body(jj, carry):
            r = 2 * jj
            @pl.when(jj > 0)
            def _():
                wait_s(r1, ss1, r - 1)
            pltpu.async_copy(y_hbm.at[sidx.at[r + 1]], r1, sg1)
            wait_g(r0, sg0, r)
            pltpu.async_copy(r0, acc.at[didx.at[r]], ss0, add=True)
            wait_g(r1, sg1, r + 1)
            pltpu.async_copy(r1, acc.at[didx.at[r + 1]], ss1, add=True)
            wait_s(r0, ss0, r)
            @pl.when(r + 2 < n)
            def _():
                pltpu.async_copy(y_hbm.at[sidx.at[r + 2]], r0, sg0)
            return carry

        lax.fori_loop(0, n // 2, body, 0)
        if n % 2:
            wait_s(r1, ss1, n - 2)
            wait_g(r0, sg0, n - 1)
            pltpu.async_copy(r0, acc.at[didx.at[n - 1]], ss0, add=True)
            wait_s(r0, ss0, n - 1)
        else:
            wait_s(r1, ss1, n - 1)

    run(_NCHA)
    pltpu.sync_copy(src_hbm.at[wid, pl.ds(_NCHA, _NCHT - _NCHA)], sidx)
    pltpu.sync_copy(dst_hbm.at[wid, pl.ds(_NCHA, _NCHT - _NCHA)], didx)
    run(_NCHT - _NCHA)

    plsc.subcore_barrier()
    sizes = [_CH] * nz + ([rem] if rem else [])
    bufs, wsems = [r0, r1], [sg0, sg1]
    for k, sz in enumerate(sizes):
        b, ws = bufs[k % 2], wsems[k % 2]
        off = s * _RPT + k * _CH
        if k >= 2:
            prev = sizes[k - 2]
            pltpu.make_async_copy(
                bufs[k % 2].at[pl.ds(0, prev)],
                out_hbm.at[pl.ds(c * _NPAD + s * _RPT + (k - 2) * _CH, prev)],
                ws).wait()
        pltpu.sync_copy(acc.at[pl.ds(off, sz)], b.at[pl.ds(0, sz)])
        pltpu.async_copy(b.at[pl.ds(0, sz)],
                         out_hbm.at[pl.ds(c * _NPAD + off, sz)], ws)
    for k in range(max(0, len(sizes) - 2), len(sizes)):
        sz = sizes[k]
        pltpu.make_async_copy(
            bufs[k % 2].at[pl.ds(0, sz)],
            out_hbm.at[pl.ds(c * _NPAD + s * _RPT + k * _CH, sz)],
            wsems[k % 2]).wait()


@functools.partial(
    pl.kernel,
    mesh=_mesh(),
    out_type=jax.ShapeDtypeStruct((_NC * _NPAD, _D), jnp.float32),
    scratch_types=[
        pltpu.VMEM((_NCHT - _NCHA, _CH), jnp.int32),
        pltpu.VMEM((_NCHT - _NCHA, _CH), jnp.int32),
        pltpu.VMEM((_CH, _D), jnp.float32),
        pltpu.VMEM((_CH, _D), jnp.float32),
        pltpu.VMEM_SHARED((_NPAD, _D), jnp.float32),
        pltpu.SemaphoreType.DMA,
        pltpu.SemaphoreType.DMA,
        pltpu.SemaphoreType.DMA,
        pltpu.SemaphoreType.DMA,
    ],
)
def _agg_sc(y_hbm, src_hbm, dst_hbm, zero_hbm, out_hbm,
            sidx, didx, r0, r1, acc, sg0, sg1, ss0, ss1):
    _agg_body(y_hbm, src_hbm, dst_hbm, zero_hbm, out_hbm,
              sidx, didx, r0, r1, acc, sg0, sg1, ss0, ss1)


_BR = 5056


def _xw_body(x_ref, w_ref, xw_ref):
    xw_ref[...] = jnp.dot(x_ref[...], w_ref[...],
                          preferred_element_type=jnp.float32)


def _tc_xw(x, W1):
    return pl.pallas_call(
        _xw_body,
        grid=(_NPAD // _BR,),
        in_specs=[
            pl.BlockSpec((_BR, _D), lambda i: (i, 0)),
            pl.BlockSpec((_D, _D), lambda i: (0, 0)),
        ],
        out_specs=pl.BlockSpec((_BR, _D), lambda i: (i, 0)),
        out_shape=jax.ShapeDtypeStruct((_NPAD, _D), jnp.float32),
    )(x, W1)


def _scale_body(xw_ref, d0_ref, d1_ref, y_ref, dinv_ref):
    deg = d0_ref[0] + d1_ref[0]
    dinv_row = lax.rsqrt(jnp.maximum(deg, 1e-12))
    dinv = jnp.transpose(dinv_row)
    dinv_ref[...] = dinv
    y_ref[...] = xw_ref[...] * dinv


def _tc_scale(xw, degr):
    return pl.pallas_call(
        _scale_body,
        grid=(_NPAD // _BR,),
        in_specs=[
            pl.BlockSpec((_BR, _D), lambda i: (i, 0)),
            pl.BlockSpec((1, 1, _BR), lambda i: (i, 0, 0)),
            pl.BlockSpec((1, 1, _BR), lambda i: (i + _NPAD // _BR, 0, 0)),
        ],
        out_specs=[
            pl.BlockSpec((_BR, _D), lambda i: (i, 0)),
            pl.BlockSpec((_BR, 1), lambda i: (i, 0)),
        ],
        out_shape=[
            jax.ShapeDtypeStruct((_NPAD, _D), jnp.float32),
            jax.ShapeDtypeStruct((_NPAD, 1), jnp.float32),
        ],
    )(xw, degr, degr)


def _mid_body(a_ref, dinv_ref, b1_ref, w2_ref, y2_ref):
    dinv = dinv_ref[...]
    h1 = jnp.maximum((a_ref[0] + a_ref[1]) * dinv + b1_ref[...], 0.0)
    y2_ref[...] = jnp.dot(h1, w2_ref[...],
                          preferred_element_type=jnp.float32) * dinv


def _tc_mid(a, dinv, b1, W2):
    return pl.pallas_call(
        _mid_body,
        grid=(_NPAD // _BR,),
        in_specs=[
            pl.BlockSpec((2, _BR, _D), lambda i: (0, i, 0)),
            pl.BlockSpec((_BR, 1), lambda i: (i, 0)),
            pl.BlockSpec((1, _D), lambda i: (0, 0)),
            pl.BlockSpec((_D, _D), lambda i: (0, 0)),
        ],
        out_specs=pl.BlockSpec((_BR, _D), lambda i: (i, 0)),
        out_shape=jax.ShapeDtypeStruct((_NPAD, _D), jnp.float32),
    )(a, dinv, b1, W2)


def _fin_body(a_ref, dinv_ref, b2_ref, p1w_ref, p1b_ref,
              p2w_ref, p2b_ref, z_ref, h_ref):
    dinv = dinv_ref[...]
    z = (a_ref[0] + a_ref[1]) * dinv + b2_ref[...]
    z_ref[...] = z
    t = jnp.maximum(
        jnp.dot(z, p1w_ref[...], preferred_element_type=jnp.float32)
        + p1b_ref[...], 0.0)
    h_ref[...] = jnp.dot(t, p2w_ref[...],
                         preferred_element_type=jnp.float32) + p2b_ref[...]


def _tc_fin(a, dinv, b2, P1w, P1b, P2w, P2b):
    return pl.pallas_call(
        _fin_body,
        grid=(_NPAD // _BR,),
        in_specs=[
            pl.BlockSpec((2, _BR, _D), lambda i: (0, i, 0)),
            pl.BlockSpec((_BR, 1), lambda i: (i, 0)),
            pl.BlockSpec((1, _D), lambda i: (0, 0)),
            pl.BlockSpec((_D, _D), lambda i: (0, 0)),
            pl.BlockSpec((1, _D), lambda i: (0, 0)),
            pl.BlockSpec((_D, _DP), lambda i: (0, 0)),
            pl.BlockSpec((1, _DP), lambda i: (0, 0)),
        ],
        out_specs=[
            pl.BlockSpec((_BR, _D), lambda i: (i, 0)),
            pl.BlockSpec((_BR, _DP), lambda i: (i, 0)),
        ],
        out_shape=[
            jax.ShapeDtypeStruct((_N, _D), jnp.float32),
            jax.ShapeDtypeStruct((_N, _DP), jnp.float32),
        ],
    )(a, dinv, b2, P1w, P1b, P2w, P2b)


def kernel(x, edge_index, W1, b1, W2, b2, P1w, P1b, P2w, P2b):
    src = edge_index[0]
    dst = edge_index[1]
    dstp = jnp.concatenate([dst, _TAIL_DST]).reshape(32, _NCHT, _CH)

    degp = _deg_sc(dstp, _ZERO1)
    xw1 = _tc_xw(x, W1)
    y1, dinv = _tc_scale(xw1, degp.reshape(_NC * _NPAD // _BR, 1, _BR))

    src_g = lax.optimization_barrier((src, degp))[0]
    srcp = jnp.concatenate([src_g, _TAIL_SRC]).reshape(32, _NCHT, _CH)

    agg1 = _agg_sc(y1, srcp, dstp, _ZERO2).reshape(_NC, _NPAD, _D)
    y2 = _tc_mid(agg1, dinv, b1.reshape(1, _D), W2)

    agg2 = _agg_sc(y2, srcp, dstp, _ZERO2).reshape(_NC, _NPAD, _D)
    z, h = _tc_fin(agg2, dinv, b2.reshape(1, _D),
                   P1w, P1b.reshape(1, _D), P2w, P2b.reshape(1, _DP))
    return (z, h)


# speedup vs baseline: 1.2640x; 1.2640x over previous
import functools

import jax
import jax.numpy as jnp
import numpy as np
from jax import lax
from jax.experimental import pallas as pl
from jax.experimental.pallas import tpu as pltpu
from jax.experimental.pallas import tpu_sc as plsc

_N = 10000
_E = 320000
_D = 128
_DP = 64
_NPAD = 10112
_EPAD = 331776
_EPW = _EPAD // 32
_CH = 128
_NCHT = _EPW // _CH
_NCHA = 40
_RPT = _NPAD // 16

_NC, _NS = 2, 16

_NDUM = _EPAD - _E - _N
_TAIL_SRC = np.concatenate([
    np.arange(_N, dtype=np.int32),
    (np.arange(_NDUM, dtype=np.int32) * 7) % _N,
])
_TAIL_DST = np.concatenate([
    np.arange(_N, dtype=np.int32),
    _N + np.arange(_NDUM, dtype=np.int32) % (_NPAD - _N),
])
_ZERO1 = np.zeros((_NPAD,), np.float32)
_ZERO2 = np.zeros((_NPAD, _D), np.float32)


def _mesh():
    return plsc.VectorSubcoreMesh(core_axis_name="c", subcore_axis_name="s")


def _deg_body(dst_hbm, zero_hbm, out_hbm, didx, ones_v, stg, dacc, sem):
    c = lax.axis_index("c")
    s = lax.axis_index("s")
    wid = c * _NS + s
    pltpu.sync_copy(zero_hbm.at[pl.ds(s * _RPT, _RPT)], stg)
    pltpu.sync_copy(stg, dacc.at[pl.ds(s * _RPT, _RPT)])
    pltpu.sync_copy(dst_hbm.at[wid], didx)
    for i in range(_CH // 16):
        ones_v[pl.ds(i * 16, 16)] = jnp.ones((16,), jnp.float32)
    plsc.subcore_barrier()

    def body(j, carry):
        pltpu.async_copy(ones_v, dacc.at[didx.at[j]], sem, add=True)
        return carry

    lax.fori_loop(0, _NCHT, body, 0)

    def drain(j, carry):
        pltpu.make_async_copy(ones_v, dacc.at[didx.at[j]], sem).wait()
        return carry

    lax.fori_loop(0, _NCHT, drain, 0)
    plsc.subcore_barrier()
    pltpu.sync_copy(dacc.at[pl.ds(s * _RPT, _RPT)], stg)
    pltpu.sync_copy(stg, out_hbm.at[pl.ds(c * _NPAD + s * _RPT, _RPT)])


@functools.partial(
    pl.kernel,
    mesh=_mesh(),
    out_type=jax.ShapeDtypeStruct((_NC * _NPAD,), jnp.float32),
    scratch_types=[
        pltpu.VMEM((_NCHT, _CH), jnp.int32),
        pltpu.VMEM((_CH,), jnp.float32),
        pltpu.VMEM((_RPT,), jnp.float32),
        pltpu.VMEM_SHARED((_NPAD,), jnp.float32),
        pltpu.SemaphoreType.DMA,
    ],
)
def _deg_sc(dst_hbm, zero_hbm, out_hbm, didx, ones_v, stg, dacc, sem):
    _deg_body(dst_hbm, zero_hbm, out_hbm, didx, ones_v, stg, dacc, sem)


def _agg_body(y_hbm, src_hbm, dst_hbm, zero_hbm, out_hbm,
              sidx, didx, r0, r1, acc,
              sg0, sg1, ss0, ss1):
    c = lax.axis_index("c")
    s = lax.axis_index("s")
    wid = c * _NS + s
    pltpu.sync_copy(zero_hbm.at[pl.ds(s * _RPT, _CH)], r0)
    nz = _RPT // _CH
    rem = _RPT - nz * _CH
    for k in range(nz):
        pltpu.async_copy(r0, acc.at[pl.ds(s * _RPT + k * _CH, _CH)], sg0)
    if rem:
        pltpu.async_copy(r0.at[pl.ds(0, rem)],
                         acc.at[pl.ds(s * _RPT + nz * _CH, rem)], sg0)
    pltpu.async_copy(src_hbm.at[wid, pl.ds(0, _NCHA)],
                     sidx.at[pl.ds(0, _NCHA)], sg1)
    pltpu.async_copy(dst_hbm.at[wid, pl.ds(0, _NCHA)],
                     didx.at[pl.ds(0, _NCHA)], sg1)
    for k in range(nz):
        pltpu.make_async_copy(r0, acc.at[pl.ds(s * _RPT + k * _CH, _CH)],
                              sg0).wait()
    if rem:
        pltpu.make_async_copy(r0.at[pl.ds(0, rem)],
                              acc.at[pl.ds(s * _RPT + nz * _CH, rem)],
                              sg0).wait()
    pltpu.make_async_copy(src_hbm.at[wid, pl.ds(0, _NCHA)],
                          sidx.at[pl.ds(0, _NCHA)], sg1).wait()
    pltpu.make_async_copy(dst_hbm.at[wid, pl.ds(0, _NCHA)],
                          didx.at[pl.ds(0, _NCHA)], sg1).wait()
    plsc.subcore_barrier()

    def run(n):
        def wait_g(buf, sem, r):
            pltpu.make_async_copy(y_hbm.at[sidx.at[r]], buf, sem).wait()

        def wait_s(buf, sem, r):
            pltpu.make_async_copy(buf, acc.at[didx.at[r]], sem).wait()

        pltpu.async_copy(y_hbm.at[sidx.at[0]], r0, sg0)

        def body(jj, carry):
            r = 2 * jj
            @pl.when(jj > 0)
            def _():
                wait_s(r1, ss1, r - 1)
            pltpu.async_copy(y_hbm.at[sidx.at[r + 1]], r1, sg1)
            wait_g(r0, sg0, r)
            pltpu.async_copy(r0, acc.at[didx.at[r]], ss0, add=True)
            wait_s(r0, ss0, r)
            @pl.when(r + 2 < n)
            def _():
                pltpu.async_copy(y_hbm.at[sidx.at[r + 2]], r0, sg0)
            wait_g(r1, sg1, r + 1)
            pltpu.async_copy(r1, acc.at[didx.at[r + 1]], ss1, add=True)
            return carry

        lax.fori_loop(0, n // 2, body, 0)
        if n % 2:
            wait_s(r1, ss1, n - 2)
            wait_g(r0, sg0, n - 1)
            pltpu.async_copy(r0, acc.at[didx.at[n - 1]], ss0, add=True)
            wait_s(r0, ss0, n - 1)
        else:
            wait_s(r1, ss1, n - 1)

    run(_NCHA)
    pltpu.sync_copy(src_hbm.at[wid, pl.ds(_NCHA, _NCHT - _NCHA)], sidx)
    pltpu.sync_copy(dst_hbm.at[wid, pl.ds(_NCHA, _NCHT - _NCHA)], didx)
    run(_NCHT - _NCHA)

    plsc.subcore_barrier()
    sizes = [_CH] * nz + ([rem] if rem else [])
    bufs, wsems = [r0, r1], [sg0, sg1]
    for k, sz in enumerate(sizes):
        b, ws = bufs[k % 2], wsems[k % 2]
        off = s * _RPT + k * _CH
        if k >= 2:
            prev = sizes[k - 2]
            pltpu.make_async_copy(
                bufs[k % 2].at[pl.ds(0, prev)],
                out_hbm.at[pl.ds(c * _NPAD + s * _RPT + (k - 2) * _CH, prev)],
                ws).wait()
        pltpu.sync_copy(acc.at[pl.ds(off, sz)], b.at[pl.ds(0, sz)])
        pltpu.async_copy(b.at[pl.ds(0, sz)],
                         out_hbm.at[pl.ds(c * _NPAD + off, sz)], ws)
    for k in range(max(0, len(sizes) - 2), len(sizes)):
        sz = sizes[k]
        pltpu.make_async_copy(
            bufs[k % 2].at[pl.ds(0, sz)],
            out_hbm.at[pl.ds(c * _NPAD + s * _RPT + k * _CH, sz)],
            wsems[k % 2]).wait()


@functools.partial(
    pl.kernel,
    mesh=_mesh(),
    out_type=jax.ShapeDtypeStruct((_NC * _NPAD, _D), jnp.float32),
    scratch_types=[
        pltpu.VMEM((_NCHT - _NCHA, _CH), jnp.int32),
        pltpu.VMEM((_NCHT - _NCHA, _CH), jnp.int32),
        pltpu.VMEM((_CH, _D), jnp.float32),
        pltpu.VMEM((_CH, _D), jnp.float32),
        pltpu.VMEM_SHARED((_NPAD, _D), jnp.float32),
        pltpu.SemaphoreType.DMA,
        pltpu.SemaphoreType.DMA,
        pltpu.SemaphoreType.DMA,
        pltpu.SemaphoreType.DMA,
    ],
)
def _agg_sc(y_hbm, src_hbm, dst_hbm, zero_hbm, out_hbm,
            sidx, didx, r0, r1, acc, sg0, sg1, ss0, ss1):
    _agg_body(y_hbm, src_hbm, dst_hbm, zero_hbm, out_hbm,
              sidx, didx, r0, r1, acc, sg0, sg1, ss0, ss1)


_BR = 5056


def _xw_body(x_ref, w_ref, xw_ref):
    xw_ref[...] = jnp.dot(x_ref[...], w_ref[...],
                          preferred_element_type=jnp.float32)


def _tc_xw(x, W1):
    return pl.pallas_call(
        _xw_body,
        grid=(_NPAD // _BR,),
        in_specs=[
            pl.BlockSpec((_BR, _D), lambda i: (i, 0)),
            pl.BlockSpec((_D, _D), lambda i: (0, 0)),
        ],
        out_specs=pl.BlockSpec((_BR, _D), lambda i: (i, 0)),
        out_shape=jax.ShapeDtypeStruct((_NPAD, _D), jnp.float32),
    )(x, W1)


def _scale_body(xw_ref, d0_ref, d1_ref, y_ref, dinv_ref):
    deg = d0_ref[0] + d1_ref[0]
    dinv_row = lax.rsqrt(jnp.maximum(deg, 1e-12))
    dinv = jnp.transpose(dinv_row)
    dinv_ref[...] = dinv
    y_ref[...] = xw_ref[...] * dinv


def _tc_scale(xw, degr):
    return pl.pallas_call(
        _scale_body,
        grid=(_NPAD // _BR,),
        in_specs=[
            pl.BlockSpec((_BR, _D), lambda i: (i, 0)),
            pl.BlockSpec((1, 1, _BR), lambda i: (i, 0, 0)),
            pl.BlockSpec((1, 1, _BR), lambda i: (i + _NPAD // _BR, 0, 0)),
        ],
        out_specs=[
            pl.BlockSpec((_BR, _D), lambda i: (i, 0)),
            pl.BlockSpec((_BR, 1), lambda i: (i, 0)),
        ],
        out_shape=[
            jax.ShapeDtypeStruct((_NPAD, _D), jnp.float32),
            jax.ShapeDtypeStruct((_NPAD, 1), jnp.float32),
        ],
    )(xw, degr, degr)


def _mid_body(a_ref, dinv_ref, b1_ref, w2_ref, y2_ref):
    dinv = dinv_ref[...]
    h1 = jnp.maximum((a_ref[0] + a_ref[1]) * dinv + b1_ref[...], 0.0)
    y2_ref[...] = jnp.dot(h1, w2_ref[...],
                          preferred_element_type=jnp.float32) * dinv


def _tc_mid(a, dinv, b1, W2):
    return pl.pallas_call(
        _mid_body,
        grid=(_NPAD // _BR,),
        in_specs=[
            pl.BlockSpec((2, _BR, _D), lambda i: (0, i, 0)),
            pl.BlockSpec((_BR, 1), lambda i: (i, 0)),
            pl.BlockSpec((1, _D), lambda i: (0, 0)),
            pl.BlockSpec((_D, _D), lambda i: (0, 0)),
        ],
        out_specs=pl.BlockSpec((_BR, _D), lambda i: (i, 0)),
        out_shape=jax.ShapeDtypeStruct((_NPAD, _D), jnp.float32),
    )(a, dinv, b1, W2)


def _fin_body(a_ref, dinv_ref, b2_ref, p1w_ref, p1b_ref,
              p2w_ref, p2b_ref, z_ref, h_ref):
    dinv = dinv_ref[...]
    z = (a_ref[0] + a_ref[1]) * dinv + b2_ref[...]
    z_ref[...] = z
    t = jnp.maximum(
        jnp.dot(z, p1w_ref[...], preferred_element_type=jnp.float32)
        + p1b_ref[...], 0.0)
    h_ref[...] = jnp.dot(t, p2w_ref[...],
                         preferred_element_type=jnp.float32) + p2b_ref[...]


def _tc_fin(a, dinv, b2, P1w, P1b, P2w, P2b):
    return pl.pallas_call(
        _fin_body,
        grid=(_NPAD // _BR,),
        in_specs=[
            pl.BlockSpec((2, _BR, _D), lambda i: (0, i, 0)),
            pl.BlockSpec((_BR, 1), lambda i: (i, 0)),
            pl.BlockSpec((1, _D), lambda i: (0, 0)),
            pl.BlockSpec((_D, _D), lambda i: (0, 0)),
            pl.BlockSpec((1, _D), lambda i: (0, 0)),
            pl.BlockSpec((_D, _DP), lambda i: (0, 0)),
            pl.BlockSpec((1, _DP), lambda i: (0, 0)),
        ],
        out_specs=[
            pl.BlockSpec((_BR, _D), lambda i: (i, 0)),
            pl.BlockSpec((_BR, _DP), lambda i: (i, 0)),
        ],
        out_shape=[
            jax.ShapeDtypeStruct((_N, _D), jnp.float32),
            jax.ShapeDtypeStruct((_N, _DP), jnp.float32),
        ],
    )(a, dinv, b2, P1w, P1b, P2w, P2b)


def kernel(x, edge_index, W1, b1, W2, b2, P1w, P1b, P2w, P2b):
    src = edge_index[0]
    dst = edge_index[1]
    dstp = jnp.concatenate([dst, _TAIL_DST]).reshape(32, _NCHT, _CH)

    degp = _deg_sc(dstp, _ZERO1)
    xw1 = _tc_xw(x, W1)
    y1, dinv = _tc_scale(xw1, degp.reshape(_NC * _NPAD // _BR, 1, _BR))

    src_g = lax.optimization_barrier((src, degp))[0]
    srcp = jnp.concatenate([src_g, _TAIL_SRC]).reshape(32, _NCHT, _CH)

    agg1 = _agg_sc(y1, srcp, dstp, _ZERO2).reshape(_NC, _NPAD, _D)
    y2 = _tc_mid(agg1, dinv, b1.reshape(1, _D), W2)

    agg2 = _agg_sc(y2, srcp, dstp, _ZERO2).reshape(_NC, _NPAD, _D)
    z, h = _tc_fin(agg2, dinv, b2.reshape(1, _D),
                   P1w, P1b.reshape(1, _D), P2w, P2b.reshape(1, _DP))
    return (z, h)
